# baseline (speedup 1.0000x reference)
import jax
import jax.numpy as jnp
from jax import lax
from jax.experimental import pallas as pl
from jax.experimental.pallas import tpu as pltpu
from jax.experimental.pallas import tpu_sc as plsc

N_CORES = 2
N_SUBCORES = 16
NW = N_CORES * N_SUBCORES

BATCH = 16384
EMB = 64
HID = 768
ROW = 2 * EMB
NUM_E = 1000000
NUM_P = NUM_E // 2

B_PER_W = BATCH // NW
CHUNK = 128
N_CHUNKS = B_PER_W // CHUNK

TP_BLK = 16384
HALF_BLK = TP_BLK // 2
TP_SHIFT = TP_BLK.bit_length() - 1
HB_SHIFT = HALF_BLK.bit_length() - 1
TP_GRID = (NUM_E + TP_BLK - 1) // TP_BLK
NUM_PACKED = TP_GRID * HALF_BLK


def _pack_body(et_ref, rt_ref, eout_ref, rout_ref):
    for src, dst in ((et_ref, eout_ref), (rt_ref, rout_ref)):
        x = src[...]
        x128 = jnp.concatenate([x[:, :HALF_BLK], x[:, HALF_BLK:]], axis=0)
        z = jnp.transpose(x128, (1, 0))
        dst[...] = pltpu.bitcast(z.astype(jnp.bfloat16), jnp.float32)


def _pack(ent_t, rel_t):
    in_spec = pl.BlockSpec((EMB, TP_BLK), lambda i: (0, i))
    out_spec = pl.BlockSpec((HALF_BLK // 2, ROW), lambda i: (i, 0))
    return pl.pallas_call(
        _pack_body,
        grid=(TP_GRID,),
        in_specs=[in_spec, in_spec],
        out_specs=[out_spec, out_spec],
        out_shape=(jax.ShapeDtypeStruct((NUM_PACKED // 2, ROW),
                                        jnp.float32),) * 2,
    )(ent_t, rel_t)


def _gather_body(h_idx, r_idx, t_idx, ent_tab, rel_tab,
                 out_h, out_r, out_t, idx0, idx1, idx2, rows_v, gsem, wsem):
    wid = lax.axis_index("s") * N_CORES + lax.axis_index("c")
    base = wid * B_PER_W
    sets = ((h_idx, ent_tab, out_h, idx0),
            (r_idx, rel_tab, out_r, idx1),
            (t_idx, ent_tab, out_t, idx2))
    for idx_hbm, _, _, idx_v in sets:
        pltpu.sync_copy(idx_hbm.at[pl.ds(base, B_PER_W)], idx_v)
        for g in range(B_PER_W // 16):
            e = idx_v[pl.ds(g * 16, 16)]
            idx_v[pl.ds(g * 16, 16)] = (
                (e >> TP_SHIFT) * (HALF_BLK // 2)
                + ((e & (HALF_BLK - 1)) >> 1))
    wb = [None, None]
    for k in range(3 * N_CHUNKS):
        s, c = divmod(k, N_CHUNKS)
        slot = k % 2
        if wb[slot] is not None:
            wb[slot].wait()
        _, table, out_hbm, idx_v = sets[s]
        pltpu.async_copy(table.at[idx_v.at[pl.ds(c * CHUNK, CHUNK)]],
                         rows_v.at[slot], gsem).wait()
        wb[slot] = pltpu.async_copy(
            rows_v.at[slot], out_hbm.at[pl.ds(base + c * CHUNK, CHUNK)], wsem)
    wb[0].wait()
    wb[1].wait()


_gather = pl.kernel(
    _gather_body,
    out_type=(jax.ShapeDtypeStruct((BATCH, ROW), jnp.float32),) * 3,
    mesh=plsc.VectorSubcoreMesh(core_axis_name="c", subcore_axis_name="s",
                                num_cores=N_CORES, num_subcores=N_SUBCORES),
    scratch_types=[
        pltpu.VMEM((B_PER_W,), jnp.int32),
        pltpu.VMEM((B_PER_W,), jnp.int32),
        pltpu.VMEM((B_PER_W,), jnp.int32),
        pltpu.VMEM((2, CHUNK, ROW), jnp.float32),
        pltpu.SemaphoreType.DMA,
        pltpu.SemaphoreType.DMA,
    ],
)

MM_BLK = 2048


def _mm_body(h_ref, r_ref, t_ref, eh_ref, er_ref, et_ref, w_ref, b_ref,
             oh_ref, or_ref, ot_ref):
    w2 = w_ref[...]
    bias = b_ref[...]
    col_hi = (lax.broadcasted_iota(jnp.int32, (MM_BLK, ROW), 1) >= EMB)
    for x_ref, e_ref, o_ref in ((h_ref, eh_ref, oh_ref),
                                (r_ref, er_ref, or_ref),
                                (t_ref, et_ref, ot_ref)):
        e = e_ref[...]
        xi = pltpu.bitcast(x_ref[...], jnp.int32)
        even = pltpu.bitcast(xi << 16, jnp.float32)
        odd = pltpu.bitcast(xi & jnp.int32(-65536), jnp.float32)
        x = jnp.where((e & 1) != 0, odd, even)
        hi = ((e >> HB_SHIFT) & 1) != 0
        keep = jnp.where(col_hi == hi, 1.0, 0.0)
        o_ref[...] = jnp.dot(x * keep, w2,
                             preferred_element_type=jnp.float32) + bias


def _project(h_rows, r_rows, t_rows, eh, er, et, W2, b2):
    row_spec = pl.BlockSpec((MM_BLK, ROW), lambda i: (i, 0))
    par_spec = pl.BlockSpec((MM_BLK, 1), lambda i: (i, 0))
    out_spec = pl.BlockSpec((MM_BLK, HID), lambda i: (i, 0))
    return pl.pallas_call(
        _mm_body,
        grid=(BATCH // MM_BLK,),
        in_specs=[
            row_spec, row_spec, row_spec,
            par_spec, par_spec, par_spec,
            pl.BlockSpec((ROW, HID), lambda i: (0, 0)),
            pl.BlockSpec((1, HID), lambda i: (0, 0)),
        ],
        out_specs=[out_spec, out_spec, out_spec],
        out_shape=(jax.ShapeDtypeStruct((BATCH, HID), jnp.float32),) * 3,
    )(h_rows, r_rows, t_rows, eh, er, et, W2, b2)


@jax.jit
def kernel(triples, ent_emb, rel_emb, W, b):
    heads, rels, tails = triples[:, 0], triples[:, 1], triples[:, 2]
    ent2, rel2 = _pack(jnp.transpose(ent_emb), jnp.transpose(rel_emb))
    h_rows, r_rows, t_rows = _gather(heads, rels, tails, ent2, rel2)
    W2 = jnp.concatenate([W, W], axis=0)
    return _project(h_rows, r_rows, t_rows,
                    heads.reshape(BATCH, 1), rels.reshape(BATCH, 1),
                    tails.reshape(BATCH, 1), W2, b.reshape(1, HID))


# speedup vs baseline: 1.0036x; 1.0036x over previous
import jax
import jax.numpy as jnp
from jax import lax
from jax.experimental import pallas as pl
from jax.experimental.pallas import tpu as pltpu
from jax.experimental.pallas import tpu_sc as plsc

N_CORES = 2
N_SUBCORES = 16
NW = N_CORES * N_SUBCORES

BATCH = 16384
EMB = 64
HID = 768
ROW = 2 * EMB
NUM_E = 1000000
NUM_P = NUM_E // 2

B_PER_W = BATCH // NW
CHUNK = 128
N_CHUNKS = B_PER_W // CHUNK

TP_BLK = 16384
HALF_BLK = TP_BLK // 2
TP_SHIFT = TP_BLK.bit_length() - 1
HB_SHIFT = HALF_BLK.bit_length() - 1
TP_GRID = (NUM_E + TP_BLK - 1) // TP_BLK
NUM_PACKED = TP_GRID * HALF_BLK


def _pack_body(et_ref, rt_ref, eout_ref, rout_ref):
    for src, dst in ((et_ref, eout_ref), (rt_ref, rout_ref)):
        x = src[...]
        x128 = jnp.concatenate([x[:, :HALF_BLK], x[:, HALF_BLK:]], axis=0)
        z = jnp.transpose(x128, (1, 0))
        dst[...] = pltpu.bitcast(z.astype(jnp.bfloat16), jnp.float32)


def _pack(ent_t, rel_t):
    in_spec = pl.BlockSpec((EMB, TP_BLK), lambda i: (0, i))
    out_spec = pl.BlockSpec((HALF_BLK // 2, ROW), lambda i: (i, 0))
    return pl.pallas_call(
        _pack_body,
        grid=(TP_GRID,),
        in_specs=[in_spec, in_spec],
        out_specs=[out_spec, out_spec],
        out_shape=(jax.ShapeDtypeStruct((NUM_PACKED // 2, ROW),
                                        jnp.float32),) * 2,
    )(ent_t, rel_t)


def _gather_body(h_idx, r_idx, t_idx, ent_tab, rel_tab,
                 out_h, out_r, out_t, idx0, idx1, idx2, rows_v, gsem, wsem):
    wid = lax.axis_index("s") * N_CORES + lax.axis_index("c")
    base = wid * B_PER_W
    sets = ((h_idx, ent_tab, out_h, idx0),
            (r_idx, rel_tab, out_r, idx1),
            (t_idx, ent_tab, out_t, idx2))
    for idx_hbm, _, _, idx_v in sets:
        pltpu.sync_copy(idx_hbm.at[pl.ds(base, B_PER_W)], idx_v)
        for g in range(B_PER_W // 16):
            e = idx_v[pl.ds(g * 16, 16)]
            idx_v[pl.ds(g * 16, 16)] = (
                (e >> TP_SHIFT) * (HALF_BLK // 2)
                + ((e & (HALF_BLK - 1)) >> 1))
    wb = [None, None]
    for k in range(3 * N_CHUNKS):
        s, c = divmod(k, N_CHUNKS)
        slot = k % 2
        if wb[slot] is not None:
            wb[slot].wait()
        _, table, out_hbm, idx_v = sets[s]
        pltpu.async_copy(table.at[idx_v.at[pl.ds(c * CHUNK, CHUNK)]],
                         rows_v.at[slot], gsem).wait()
        wb[slot] = pltpu.async_copy(
            rows_v.at[slot], out_hbm.at[pl.ds(base + c * CHUNK, CHUNK)], wsem)
    wb[0].wait()
    wb[1].wait()


_gather = pl.kernel(
    _gather_body,
    out_type=(jax.ShapeDtypeStruct((BATCH, ROW), jnp.float32),) * 3,
    mesh=plsc.VectorSubcoreMesh(core_axis_name="c", subcore_axis_name="s",
                                num_cores=N_CORES, num_subcores=N_SUBCORES),
    scratch_types=[
        pltpu.VMEM((B_PER_W,), jnp.int32),
        pltpu.VMEM((B_PER_W,), jnp.int32),
        pltpu.VMEM((B_PER_W,), jnp.int32),
        pltpu.VMEM((2, CHUNK, ROW), jnp.float32),
        pltpu.SemaphoreType.DMA,
        pltpu.SemaphoreType.DMA,
    ],
)

MM_BLK = 1024


def _mm_body(h_ref, r_ref, t_ref, eh_ref, er_ref, et_ref, w_ref, b_ref,
             oh_ref, or_ref, ot_ref):
    w2 = w_ref[...]
    bias = b_ref[...]
    col_hi = (lax.broadcasted_iota(jnp.int32, (MM_BLK, ROW), 1) >= EMB)
    for x_ref, e_ref, o_ref in ((h_ref, eh_ref, oh_ref),
                                (r_ref, er_ref, or_ref),
                                (t_ref, et_ref, ot_ref)):
        e = e_ref[...]
        xi = pltpu.bitcast(x_ref[...], jnp.int32)
        even = pltpu.bitcast(xi << 16, jnp.float32)
        odd = pltpu.bitcast(xi & jnp.int32(-65536), jnp.float32)
        x = jnp.where((e & 1) != 0, odd, even)
        hi = ((e >> HB_SHIFT) & 1) != 0
        keep = jnp.where(col_hi == hi, 1.0, 0.0)
        o_ref[...] = jnp.dot(x * keep, w2,
                             preferred_element_type=jnp.float32) + bias


def _project(h_rows, r_rows, t_rows, eh, er, et, W2, b2):
    row_spec = pl.BlockSpec((MM_BLK, ROW), lambda i: (i, 0))
    par_spec = pl.BlockSpec((MM_BLK, 1), lambda i: (i, 0))
    out_spec = pl.BlockSpec((MM_BLK, HID), lambda i: (i, 0))
    return pl.pallas_call(
        _mm_body,
        grid=(BATCH // MM_BLK,),
        in_specs=[
            row_spec, row_spec, row_spec,
            par_spec, par_spec, par_spec,
            pl.BlockSpec((ROW, HID), lambda i: (0, 0)),
            pl.BlockSpec((1, HID), lambda i: (0, 0)),
        ],
        out_specs=[out_spec, out_spec, out_spec],
        out_shape=(jax.ShapeDtypeStruct((BATCH, HID), jnp.float32),) * 3,
    )(h_rows, r_rows, t_rows, eh, er, et, W2, b2)


@jax.jit
def kernel(triples, ent_emb, rel_emb, W, b):
    heads, rels, tails = triples[:, 0], triples[:, 1], triples[:, 2]
    ent2, rel2 = _pack(jnp.transpose(ent_emb), jnp.transpose(rel_emb))
    h_rows, r_rows, t_rows = _gather(heads, rels, tails, ent2, rel2)
    W2 = jnp.concatenate([W, W], axis=0)
    return _project(h_rows, r_rows, t_rows,
                    heads.reshape(BATCH, 1), rels.reshape(BATCH, 1),
                    tails.reshape(BATCH, 1), W2, b.reshape(1, HID))
